# initial kernel scaffold (unmeasured)
import jax
import jax.numpy as jnp
from jax import lax
from jax.experimental import pallas as pl
from jax.experimental.pallas import tpu as pltpu

N_DEV = 4
SQ = 1024
HL = 8
DH = 128
WIN = 128
KV_WIN = SQ + WIN
D_LOC = HL * DH
D_MODEL = 1024
SCALE = 0.08838834764831843


def kernel(x, Wq, K_ext, V_ext, Wo):
    def body(x_ref, wq_ref, k_ref, v_ref, wo_ref, out_ref,
             kg, vg, q_bf, ctx_bf, ar_buf,
             kv_send_sems, kv_recv_sems, ar_send_sems, ar_recv_sems,
             copy_sems):
        my = lax.axis_index("i")

        bsem = pltpu.get_barrier_semaphore()
        for j in range(N_DEV):
            @pl.when(my != j)
            def _(j=j):
                pl.semaphore_signal(bsem, inc=1, device_id=(j,),
                                    device_id_type=pl.DeviceIdType.MESH)
        pl.semaphore_wait(bsem, N_DEV - 1)

        rdmas0 = []
        for jj, j in enumerate((1, 2, 3)):
            for t, (src, dst) in enumerate(((k_ref, kg), (v_ref, vg))):
                rdmas0.append(pltpu.make_async_remote_copy(
                    src_ref=src.at[0, :, j * HL:(j + 1) * HL, :],
                    dst_ref=dst.at[0:SQ],
                    send_sem=kv_send_sems.at[2 * jj + t],
                    recv_sem=kv_recv_sems.at[t],
                    device_id=(j,),
                    device_id_type=pl.DeviceIdType.MESH,
                ))
        rdmas1 = []
        for jj, j in enumerate((0, 2, 3)):
            for t, (src, dst) in enumerate(((k_ref, kg), (v_ref, vg))):
                rdmas1.append(pltpu.make_async_remote_copy(
                    src_ref=src.at[0, 0:WIN, j * HL:(j + 1) * HL, :],
                    dst_ref=dst.at[SQ:KV_WIN],
                    send_sem=kv_send_sems.at[2 * jj + t],
                    recv_sem=kv_recv_sems.at[2 + t],
                    device_id=(j,),
                    device_id_type=pl.DeviceIdType.MESH,
                ))

        @pl.when(my == 0)
        def _():
            for r in rdmas0:
                r.start()
            for t, (src, dst) in enumerate(((k_ref, kg), (v_ref, vg))):
                cp = pltpu.make_async_copy(
                    src.at[0, :, 0:HL, :], dst.at[0:SQ], copy_sems.at[t])
                cp.start()
                cp.wait()

        @pl.when(my == 1)
        def _():
            for r in rdmas1:
                r.start()
            for t, (src, dst) in enumerate(((k_ref, kg), (v_ref, vg))):
                cp = pltpu.make_async_copy(
                    src.at[0, 0:WIN, HL:2 * HL, :], dst.at[SQ:KV_WIN],
                    copy_sems.at[t])
                cp.start()
                cp.wait()

        x_bf = x_ref[0].astype(jnp.bfloat16)
        wq_bf = wq_ref[...].astype(jnp.bfloat16)
        q = jnp.dot(x_bf, wq_bf, preferred_element_type=jnp.float32)
        q_bf[...] = q.astype(jnp.bfloat16)

        @pl.when(my != 0)
        def _():
            for t, dst in ((0, kg), (1, vg)):
                recv = pltpu.make_async_remote_copy(
                    src_ref=dst.at[0:SQ], dst_ref=dst.at[0:SQ],
                    send_sem=kv_send_sems.at[t],
                    recv_sem=kv_recv_sems.at[t],
                    device_id=(0,), device_id_type=pl.DeviceIdType.MESH)
                recv.wait_recv()

        @pl.when(my != 1)
        def _():
            for t, dst in ((0, kg), (1, vg)):
                recv = pltpu.make_async_remote_copy(
                    src_ref=dst.at[SQ:KV_WIN], dst_ref=dst.at[SQ:KV_WIN],
                    send_sem=kv_send_sems.at[t],
                    recv_sem=kv_recv_sems.at[2 + t],
                    device_id=(1,), device_id_type=pl.DeviceIdType.MESH)
                recv.wait_recv()

        qi = lax.broadcasted_iota(jnp.int32, (SQ, KV_WIN), 0)
        ki = lax.broadcasted_iota(jnp.int32, (SQ, KV_WIN), 1)
        band = jnp.abs(qi - ki) <= WIN
        for h in range(HL):
            qh = q_bf[:, h * DH:(h + 1) * DH]
            kh = kg[:, h, :].astype(jnp.bfloat16)
            s = lax.dot_general(qh, kh, (((1,), (1,)), ((), ())),
                                preferred_element_type=jnp.float32) * SCALE
            s = jnp.where(band, s, -1e9)
            m = jnp.max(s, axis=1, keepdims=True)
            e = jnp.exp(s - m)
            w = (e / jnp.sum(e, axis=1, keepdims=True)).astype(jnp.bfloat16)
            vh = vg[:, h, :].astype(jnp.bfloat16)
            ctx_h = jnp.dot(w, vh, preferred_element_type=jnp.float32)
            ctx_bf[:, h * DH:(h + 1) * DH] = ctx_h.astype(jnp.bfloat16)

        wo_bf = wo_ref[...].astype(jnp.bfloat16)
        partial = jnp.dot(ctx_bf[...], wo_bf,
                          preferred_element_type=jnp.float32)

        for i in range(N_DEV):
            @pl.when(my == i)
            def _(i=i):
                ar_buf[i, :, :] = partial.astype(jnp.bfloat16)
                sends = []
                for jj, j in enumerate(p for p in range(N_DEV) if p != i):
                    r = pltpu.make_async_remote_copy(
                        src_ref=ar_buf.at[i], dst_ref=ar_buf.at[i],
                        send_sem=ar_send_sems.at[jj],
                        recv_sem=ar_recv_sems.at[i],
                        device_id=(j,), device_id_type=pl.DeviceIdType.MESH)
                    r.start()
                    sends.append(r)
                for j in (p for p in range(N_DEV) if p != i):
                    recv = pltpu.make_async_remote_copy(
                        src_ref=ar_buf.at[j], dst_ref=ar_buf.at[j],
                        send_sem=ar_send_sems.at[0],
                        recv_sem=ar_recv_sems.at[j],
                        device_id=(j,), device_id_type=pl.DeviceIdType.MESH)
                    recv.wait_recv()
                for r in sends:
                    r.wait_send()

        @pl.when(my == 0)
        def _():
            for r in rdmas0:
                r.wait_send()

        @pl.when(my == 1)
        def _():
            for r in rdmas1:
                r.wait_send()

        out_ref[0, :, :] = (ar_buf[0, :, :].astype(jnp.float32)
                            + ar_buf[1, :, :].astype(jnp.float32)
                            + ar_buf[2, :, :].astype(jnp.float32)
                            + ar_buf[3, :, :].astype(jnp.float32))

    return pl.pallas_call(
        body,
        out_shape=jax.ShapeDtypeStruct((1, SQ, D_MODEL), jnp.float32),
        in_specs=[
            pl.BlockSpec(memory_space=pltpu.VMEM),
            pl.BlockSpec(memory_space=pltpu.VMEM),
            pl.BlockSpec(memory_space=pltpu.ANY),
            pl.BlockSpec(memory_space=pltpu.ANY),
            pl.BlockSpec(memory_space=pltpu.VMEM),
        ],
        out_specs=pl.BlockSpec(memory_space=pltpu.VMEM),
        scratch_shapes=[
            pltpu.VMEM((KV_WIN, HL, DH), jnp.float32),
            pltpu.VMEM((KV_WIN, HL, DH), jnp.float32),
            pltpu.VMEM((SQ, D_LOC), jnp.bfloat16),
            pltpu.VMEM((SQ, D_LOC), jnp.bfloat16),
            pltpu.VMEM((N_DEV, SQ, D_MODEL), jnp.bfloat16),
            pltpu.SemaphoreType.DMA((6,)),
            pltpu.SemaphoreType.DMA((4,)),
            pltpu.SemaphoreType.DMA((3,)),
            pltpu.SemaphoreType.DMA((4,)),
            pltpu.SemaphoreType.DMA((2,)),
        ],
        compiler_params=pltpu.CompilerParams(collective_id=0),
    )(x, Wq, K_ext, V_ext, Wo)


# baseline (device time: 262960 ns/iter reference)
import jax
import jax.numpy as jnp
from jax import lax
from jax.experimental import pallas as pl
from jax.experimental.pallas import tpu as pltpu

N_DEV = 4
SQ = 1024
HL = 8
DH = 128
WIN = 128
KV_WIN = SQ + WIN
D_LOC = HL * DH
D_MODEL = 1024
SCALE = 0.08838834764831843


def kernel(x, Wq, K_ext, V_ext, Wo):
    def body(x_ref, wq_ref, k_ref, v_ref, wo_ref, out_ref,
             kg, vg, q_bf, ctx_bf, ar_buf,
             kv_send_sems, kv_recv_sems, ar_send_sems, ar_recv_sems,
             copy_sems):
        my = lax.axis_index("i")

        bsem = pltpu.get_barrier_semaphore()
        for j in range(N_DEV):
            @pl.when(my != j)
            def _(j=j):
                pl.semaphore_signal(bsem, inc=1, device_id=(j,),
                                    device_id_type=pl.DeviceIdType.MESH)
        pl.semaphore_wait(bsem, N_DEV - 1)

        rdmas0 = []
        for jj, j in enumerate((1, 2, 3)):
            for t, (src, dst) in enumerate(((k_ref, kg), (v_ref, vg))):
                rdmas0.append(pltpu.make_async_remote_copy(
                    src_ref=src.at[0, :, j * HL:(j + 1) * HL, :],
                    dst_ref=dst.at[0:SQ],
                    send_sem=kv_send_sems.at[2 * jj + t],
                    recv_sem=kv_recv_sems.at[t],
                    device_id=(j,),
                    device_id_type=pl.DeviceIdType.MESH,
                ))
        rdmas1 = []
        for jj, j in enumerate((0, 2, 3)):
            for t, (src, dst) in enumerate(((k_ref, kg), (v_ref, vg))):
                rdmas1.append(pltpu.make_async_remote_copy(
                    src_ref=src.at[0, 0:WIN, j * HL:(j + 1) * HL, :],
                    dst_ref=dst.at[SQ:KV_WIN],
                    send_sem=kv_send_sems.at[2 * jj + t],
                    recv_sem=kv_recv_sems.at[2 + t],
                    device_id=(j,),
                    device_id_type=pl.DeviceIdType.MESH,
                ))

        @pl.when(my == 0)
        def _():
            for r in rdmas0:
                r.start()
            for t, (src, dst) in enumerate(((k_ref, kg), (v_ref, vg))):
                cp = pltpu.make_async_copy(
                    src.at[0, :, 0:HL, :], dst.at[0:SQ], copy_sems.at[t])
                cp.start()
                cp.wait()

        @pl.when(my == 1)
        def _():
            for r in rdmas1:
                r.start()
            for t, (src, dst) in enumerate(((k_ref, kg), (v_ref, vg))):
                cp = pltpu.make_async_copy(
                    src.at[0, 0:WIN, HL:2 * HL, :], dst.at[SQ:KV_WIN],
                    copy_sems.at[t])
                cp.start()
                cp.wait()

        x_bf = x_ref[0].astype(jnp.bfloat16)
        wq_bf = wq_ref[...].astype(jnp.bfloat16)
        q = jnp.dot(x_bf, wq_bf, preferred_element_type=jnp.float32)
        q_bf[...] = q.astype(jnp.bfloat16)

        @pl.when(my != 0)
        def _():
            for t, dst in ((0, kg), (1, vg)):
                recv = pltpu.make_async_remote_copy(
                    src_ref=dst.at[0:SQ], dst_ref=dst.at[0:SQ],
                    send_sem=kv_send_sems.at[t],
                    recv_sem=kv_recv_sems.at[t],
                    device_id=(0,), device_id_type=pl.DeviceIdType.MESH)
                recv.wait_recv()

        @pl.when(my != 1)
        def _():
            for t, dst in ((0, kg), (1, vg)):
                recv = pltpu.make_async_remote_copy(
                    src_ref=dst.at[SQ:KV_WIN], dst_ref=dst.at[SQ:KV_WIN],
                    send_sem=kv_send_sems.at[t],
                    recv_sem=kv_recv_sems.at[2 + t],
                    device_id=(1,), device_id_type=pl.DeviceIdType.MESH)
                recv.wait_recv()

        qi = lax.broadcasted_iota(jnp.int32, (SQ, KV_WIN), 0)
        ki = lax.broadcasted_iota(jnp.int32, (SQ, KV_WIN), 1)
        band = jnp.abs(qi - ki) <= WIN
        for h in range(HL):
            qh = q_bf[:, h * DH:(h + 1) * DH]
            kh = kg[:, h, :].astype(jnp.bfloat16)
            s = lax.dot_general(qh, kh, (((1,), (1,)), ((), ())),
                                preferred_element_type=jnp.float32) * SCALE
            s = jnp.where(band, s, -1e9)
            m = jnp.max(s, axis=1, keepdims=True)
            e = jnp.exp(s - m)
            w = (e / jnp.sum(e, axis=1, keepdims=True)).astype(jnp.bfloat16)
            vh = vg[:, h, :].astype(jnp.bfloat16)
            ctx_h = jnp.dot(w, vh, preferred_element_type=jnp.float32)
            ctx_bf[:, h * DH:(h + 1) * DH] = ctx_h.astype(jnp.bfloat16)

        wo_bf = wo_ref[...].astype(jnp.bfloat16)
        partial = jnp.dot(ctx_bf[...], wo_bf,
                          preferred_element_type=jnp.float32)

        for i in range(N_DEV):
            @pl.when(my == i)
            def _(i=i):
                ar_buf[i, :, :] = partial.astype(jnp.bfloat16)
                sends = []
                for jj, j in enumerate(p for p in range(N_DEV) if p != i):
                    r = pltpu.make_async_remote_copy(
                        src_ref=ar_buf.at[i], dst_ref=ar_buf.at[i],
                        send_sem=ar_send_sems.at[jj],
                        recv_sem=ar_recv_sems.at[i],
                        device_id=(j,), device_id_type=pl.DeviceIdType.MESH)
                    r.start()
                    sends.append(r)
                for j in (p for p in range(N_DEV) if p != i):
                    recv = pltpu.make_async_remote_copy(
                        src_ref=ar_buf.at[j], dst_ref=ar_buf.at[j],
                        send_sem=ar_send_sems.at[0],
                        recv_sem=ar_recv_sems.at[j],
                        device_id=(j,), device_id_type=pl.DeviceIdType.MESH)
                    recv.wait_recv()
                for r in sends:
                    r.wait_send()

        @pl.when(my == 0)
        def _():
            for r in rdmas0:
                r.wait_send()

        @pl.when(my == 1)
        def _():
            for r in rdmas1:
                r.wait_send()

        out_ref[0, :, :] = (ar_buf[0, :, :].astype(jnp.float32)
                            + ar_buf[1, :, :].astype(jnp.float32)
                            + ar_buf[2, :, :].astype(jnp.float32)
                            + ar_buf[3, :, :].astype(jnp.float32))

    return pl.pallas_call(
        body,
        out_shape=jax.ShapeDtypeStruct((1, SQ, D_MODEL), jnp.float32),
        in_specs=[
            pl.BlockSpec(memory_space=pltpu.VMEM),
            pl.BlockSpec(memory_space=pltpu.VMEM),
            pl.BlockSpec(memory_space=pl.ANY),
            pl.BlockSpec(memory_space=pl.ANY),
            pl.BlockSpec(memory_space=pltpu.VMEM),
        ],
        out_specs=pl.BlockSpec(memory_space=pltpu.VMEM),
        scratch_shapes=[
            pltpu.VMEM((KV_WIN, HL, DH), jnp.float32),
            pltpu.VMEM((KV_WIN, HL, DH), jnp.float32),
            pltpu.VMEM((SQ, D_LOC), jnp.bfloat16),
            pltpu.VMEM((SQ, D_LOC), jnp.bfloat16),
            pltpu.VMEM((N_DEV, SQ, D_MODEL), jnp.bfloat16),
            pltpu.SemaphoreType.DMA((6,)),
            pltpu.SemaphoreType.DMA((4,)),
            pltpu.SemaphoreType.DMA((3,)),
            pltpu.SemaphoreType.DMA((4,)),
            pltpu.SemaphoreType.DMA((2,)),
        ],
        compiler_params=pltpu.CompilerParams(collective_id=0),
    )(x, Wq, K_ext, V_ext, Wo)


# device time: 194345 ns/iter; 1.3531x vs baseline; 1.3531x over previous
import jax
import jax.numpy as jnp
from jax import lax
from jax.experimental import pallas as pl
from jax.experimental.pallas import tpu as pltpu

N_DEV = 4
SQ = 1024
HL = 8
DH = 128
WIN = 128
KV_WIN = SQ + WIN
QB = 256
KB = 512
D_LOC = HL * DH
D_MODEL = 1024
SCALE = 0.08838834764831843


def kernel(x, Wq, K_ext, V_ext, Wo):
    def body(x_ref, wq_ref, k_ref, v_ref, wo_ref, out_ref,
             kv_bf, stag, kgvg, q_bf, ctx_bf, ar_send, ar_recv,
             kv_send_sems, kv_recv_sems, ar_send_sems, ar_recv_sems,
             copy_sems):
        my = lax.axis_index("i")

        bsem = pltpu.get_barrier_semaphore()
        for j in range(N_DEV):
            @pl.when(my != j)
            def _(j=j):
                pl.semaphore_signal(bsem, inc=1, device_id=(j,),
                                    device_id_type=pl.DeviceIdType.MESH)
        pl.semaphore_wait(bsem, N_DEV - 1)

        def make_kv_sends(sender, peers, rows):
            dst_sl = slice(0, SQ) if sender == 0 else slice(SQ, KV_WIN)
            return [pltpu.make_async_remote_copy(
                src_ref=kv_bf.at[jj, :, 0:rows],
                dst_ref=kgvg.at[:, dst_sl],
                send_sem=kv_send_sems.at[jj],
                recv_sem=kv_recv_sems.at[sender],
                device_id=(j,), device_id_type=pl.DeviceIdType.MESH)
                for jj, j in enumerate(peers)]

        kv_sends0 = make_kv_sends(0, (1, 2, 3), SQ)
        kv_sends1 = make_kv_sends(1, (0, 2, 3), WIN)

        def convert_and_send(sender, peers, rows, my_heads, sends):
            dst_sl = slice(0, SQ) if sender == 0 else slice(SQ, KV_WIN)
            for jj, j in enumerate(peers):
                for t, src in enumerate((k_ref, v_ref)):
                    cp = pltpu.make_async_copy(
                        src.at[0, 0:rows, j * HL:(j + 1) * HL, :],
                        stag.at[0, 0:rows], copy_sems.at[t])
                    cp.start()
                    cp.wait()
                    kv_bf[jj, t, 0:rows] = stag[0, 0:rows].astype(jnp.bfloat16)
                sends[jj].start()
            for t, src in enumerate((k_ref, v_ref)):
                cp = pltpu.make_async_copy(
                    src.at[0, 0:rows, my_heads * HL:(my_heads + 1) * HL, :],
                    stag.at[0, 0:rows], copy_sems.at[t])
                cp.start()
                cp.wait()
                kgvg[t, dst_sl] = stag[0, 0:rows].astype(jnp.bfloat16)

        @pl.when(my == 0)
        def _():
            convert_and_send(0, (1, 2, 3), SQ, 0, kv_sends0)

        @pl.when(my == 1)
        def _():
            convert_and_send(1, (0, 2, 3), WIN, 1, kv_sends1)

        x_bf = x_ref[0].astype(jnp.bfloat16)
        wq_bf = wq_ref[...].astype(jnp.bfloat16)
        q = jnp.dot(x_bf, wq_bf, preferred_element_type=jnp.float32)
        q_bf[...] = q.astype(jnp.bfloat16)

        @pl.when(my != 0)
        def _():
            pltpu.make_async_remote_copy(
                src_ref=kgvg.at[:, 0:SQ], dst_ref=kgvg.at[:, 0:SQ],
                send_sem=kv_send_sems.at[0],
                recv_sem=kv_recv_sems.at[0],
                device_id=(0,), device_id_type=pl.DeviceIdType.MESH,
            ).wait_recv()

        @pl.when(my != 1)
        def _():
            pltpu.make_async_remote_copy(
                src_ref=kgvg.at[:, SQ:KV_WIN], dst_ref=kgvg.at[:, SQ:KV_WIN],
                send_sem=kv_send_sems.at[0],
                recv_sem=kv_recv_sems.at[1],
                device_id=(1,), device_id_type=pl.DeviceIdType.MESH,
            ).wait_recv()

        for b in range(SQ // QB):
            lo = max(0, QB * b - WIN)
            off = QB * b - lo
            qi = lax.broadcasted_iota(jnp.int32, (QB, KB), 0) + off
            ki = lax.broadcasted_iota(jnp.int32, (QB, KB), 1)
            band = jnp.abs(qi - ki) <= WIN
            for h in range(HL):
                qh = q_bf[b * QB:(b + 1) * QB, h * DH:(h + 1) * DH]
                kh = kgvg[0, lo:lo + KB, h, :]
                s = lax.dot_general(qh, kh, (((1,), (1,)), ((), ())),
                                    preferred_element_type=jnp.float32) * SCALE
                s = jnp.where(band, s, -1e9)
                m = jnp.max(s, axis=1, keepdims=True)
                e = jnp.exp(s - m)
                w = (e / jnp.sum(e, axis=1, keepdims=True)).astype(jnp.bfloat16)
                vh = kgvg[1, lo:lo + KB, h, :]
                ctx_h = jnp.dot(w, vh, preferred_element_type=jnp.float32)
                ctx_bf[b * QB:(b + 1) * QB, h * DH:(h + 1) * DH] = (
                    ctx_h.astype(jnp.bfloat16))

        wo_bf = wo_ref[...].astype(jnp.bfloat16)
        partial = jnp.dot(ctx_bf[...], wo_bf,
                          preferred_element_type=jnp.float32)

        @pl.when(my == 0)
        def _():
            for r in kv_sends0:
                r.wait_send()

        @pl.when(my == 1)
        def _():
            for r in kv_sends1:
                r.wait_send()

        acc = partial
        for step, mask in enumerate((1, 3)):
            partner = my ^ mask
            ar_send[0] = acc.astype(jnp.bfloat16)
            r = pltpu.make_async_remote_copy(
                src_ref=ar_send.at[0], dst_ref=ar_recv.at[step],
                send_sem=ar_send_sems.at[step],
                recv_sem=ar_recv_sems.at[step],
                device_id=(partner,), device_id_type=pl.DeviceIdType.MESH)
            r.start()
            r.wait()
            acc = acc + ar_recv[step].astype(jnp.float32)

        out_ref[0, :, :] = acc

    return pl.pallas_call(
        body,
        out_shape=jax.ShapeDtypeStruct((1, SQ, D_MODEL), jnp.float32),
        in_specs=[
            pl.BlockSpec(memory_space=pltpu.VMEM),
            pl.BlockSpec(memory_space=pltpu.VMEM),
            pl.BlockSpec(memory_space=pl.ANY),
            pl.BlockSpec(memory_space=pl.ANY),
            pl.BlockSpec(memory_space=pltpu.VMEM),
        ],
        out_specs=pl.BlockSpec(memory_space=pltpu.VMEM),
        scratch_shapes=[
            pltpu.VMEM((3, 2, SQ, HL, DH), jnp.bfloat16),
            pltpu.VMEM((1, SQ, HL, DH), jnp.float32),
            pltpu.VMEM((2, KV_WIN, HL, DH), jnp.bfloat16),
            pltpu.VMEM((SQ, D_LOC), jnp.bfloat16),
            pltpu.VMEM((SQ, D_LOC), jnp.bfloat16),
            pltpu.VMEM((1, SQ, D_MODEL), jnp.bfloat16),
            pltpu.VMEM((2, SQ, D_MODEL), jnp.bfloat16),
            pltpu.SemaphoreType.DMA((3,)),
            pltpu.SemaphoreType.DMA((2,)),
            pltpu.SemaphoreType.DMA((2,)),
            pltpu.SemaphoreType.DMA((2,)),
            pltpu.SemaphoreType.DMA((2,)),
        ],
        compiler_params=pltpu.CompilerParams(
            collective_id=0, vmem_limit_bytes=60 * 1024 * 1024),
    )(x, Wq, K_ext, V_ext, Wo)


# device time: 167249 ns/iter; 1.5723x vs baseline; 1.1620x over previous
import jax
import jax.numpy as jnp
from jax import lax
from jax.experimental import pallas as pl
from jax.experimental.pallas import tpu as pltpu

N_DEV = 4
SQ = 1024
HL = 8
DH = 128
WIN = 128
KV_WIN = SQ + WIN
QB = 256
KB = 512
D_LOC = HL * DH
D_MODEL = 1024
SCALE = 0.08838834764831843


def kernel(x, Wq, K_ext, V_ext, Wo):
    def body(x_ref, wq_ref, k_ref, v_ref, wo_ref, out_ref,
             kv_bf, stag, kgvg, q_bf, ctx_bf, ar_send, ar_recv, relay_buf,
             kv_send_sems, kv_recv_sems, ar_send_sems, ar_recv_sems,
             copy_sems, relay_sems, fwd_send_sems):
        my = lax.axis_index("i")

        bsem = pltpu.get_barrier_semaphore()
        for j in range(N_DEV):
            @pl.when(my != j)
            def _(j=j):
                pl.semaphore_signal(bsem, inc=1, device_id=(j,),
                                    device_id_type=pl.DeviceIdType.MESH)
        pl.semaphore_wait(bsem, N_DEV - 1)

        def convert(slot, rows, heads0):
            for t, src in enumerate((k_ref, v_ref)):
                cp = pltpu.make_async_copy(
                    src.at[0, 0:rows, heads0:heads0 + HL, :],
                    stag.at[0, 0:rows], copy_sems.at[t])
                cp.start()
                cp.wait()
                kv_bf[slot, t, 0:rows] = stag[0, 0:rows].astype(jnp.bfloat16)

        def local_fill(rows, my_heads, dst_sl):
            for t, src in enumerate((k_ref, v_ref)):
                cp = pltpu.make_async_copy(
                    src.at[0, 0:rows, my_heads * HL:(my_heads + 1) * HL, :],
                    stag.at[0, 0:rows], copy_sems.at[t])
                cp.start()
                cp.wait()
                kgvg[t, dst_sl] = stag[0, 0:rows].astype(jnp.bfloat16)

        kv_sends0 = [
            pltpu.make_async_remote_copy(
                src_ref=kv_bf.at[2, 0], dst_ref=relay_buf,
                send_sem=kv_send_sems.at[2], recv_sem=relay_sems.at[0],
                device_id=(1,), device_id_type=pl.DeviceIdType.MESH),
            pltpu.make_async_remote_copy(
                src_ref=kv_bf.at[2, 1], dst_ref=relay_buf,
                send_sem=kv_send_sems.at[3], recv_sem=relay_sems.at[0],
                device_id=(3,), device_id_type=pl.DeviceIdType.MESH),
            pltpu.make_async_remote_copy(
                src_ref=kv_bf.at[0], dst_ref=kgvg.at[:, 0:SQ],
                send_sem=kv_send_sems.at[0], recv_sem=kv_recv_sems.at[0],
                device_id=(1,), device_id_type=pl.DeviceIdType.MESH),
            pltpu.make_async_remote_copy(
                src_ref=kv_bf.at[1], dst_ref=kgvg.at[:, 0:SQ],
                send_sem=kv_send_sems.at[1], recv_sem=kv_recv_sems.at[0],
                device_id=(3,), device_id_type=pl.DeviceIdType.MESH),
        ]
        kv_sends1 = [pltpu.make_async_remote_copy(
            src_ref=kv_bf.at[jj, :, 0:WIN],
            dst_ref=kgvg.at[:, SQ:KV_WIN],
            send_sem=kv_send_sems.at[jj],
            recv_sem=kv_recv_sems.at[1],
            device_id=(j,), device_id_type=pl.DeviceIdType.MESH)
            for jj, j in enumerate((0, 2, 3))]
        fwd1 = pltpu.make_async_remote_copy(
            src_ref=relay_buf, dst_ref=kgvg.at[0, 0:SQ],
            send_sem=fwd_send_sems.at[0], recv_sem=kv_recv_sems.at[2],
            device_id=(2,), device_id_type=pl.DeviceIdType.MESH)
        fwd3 = pltpu.make_async_remote_copy(
            src_ref=relay_buf, dst_ref=kgvg.at[1, 0:SQ],
            send_sem=fwd_send_sems.at[0], recv_sem=kv_recv_sems.at[3],
            device_id=(2,), device_id_type=pl.DeviceIdType.MESH)

        @pl.when(my == 0)
        def _():
            convert(2, SQ, 2 * HL)
            kv_sends0[0].start()
            kv_sends0[1].start()
            convert(0, SQ, 1 * HL)
            kv_sends0[2].start()
            convert(1, SQ, 3 * HL)
            kv_sends0[3].start()
            local_fill(SQ, 0, slice(0, SQ))

        @pl.when(my == 1)
        def _():
            for jj, j in enumerate((0, 2, 3)):
                convert(jj, WIN, j * HL)
                kv_sends1[jj].start()
            local_fill(WIN, 1, slice(SQ, KV_WIN))

        x_bf = x_ref[0].astype(jnp.bfloat16)
        wq_bf = wq_ref[...].astype(jnp.bfloat16)
        q = jnp.dot(x_bf, wq_bf, preferred_element_type=jnp.float32)
        q_bf[...] = q.astype(jnp.bfloat16)

        @pl.when((my == 1) | (my == 3))
        def _():
            pltpu.make_async_remote_copy(
                src_ref=relay_buf, dst_ref=relay_buf,
                send_sem=fwd_send_sems.at[0],
                recv_sem=relay_sems.at[0],
                device_id=(0,), device_id_type=pl.DeviceIdType.MESH,
            ).wait_recv()

        @pl.when(my == 1)
        def _():
            fwd1.start()

        @pl.when(my == 3)
        def _():
            fwd3.start()

        @pl.when((my == 1) | (my == 3))
        def _():
            pltpu.make_async_remote_copy(
                src_ref=kgvg.at[:, 0:SQ], dst_ref=kgvg.at[:, 0:SQ],
                send_sem=kv_send_sems.at[0],
                recv_sem=kv_recv_sems.at[0],
                device_id=(0,), device_id_type=pl.DeviceIdType.MESH,
            ).wait_recv()

        @pl.when(my == 2)
        def _():
            for t in (2, 3):
                pltpu.make_async_remote_copy(
                    src_ref=kgvg.at[t - 2, 0:SQ], dst_ref=kgvg.at[t - 2, 0:SQ],
                    send_sem=kv_send_sems.at[0],
                    recv_sem=kv_recv_sems.at[t],
                    device_id=(0,), device_id_type=pl.DeviceIdType.MESH,
                ).wait_recv()

        @pl.when(my != 1)
        def _():
            pltpu.make_async_remote_copy(
                src_ref=kgvg.at[:, SQ:KV_WIN], dst_ref=kgvg.at[:, SQ:KV_WIN],
                send_sem=kv_send_sems.at[0],
                recv_sem=kv_recv_sems.at[1],
                device_id=(1,), device_id_type=pl.DeviceIdType.MESH,
            ).wait_recv()

        for b in range(SQ // QB):
            lo = max(0, QB * b - WIN)
            off = QB * b - lo
            qi = lax.broadcasted_iota(jnp.int32, (QB, KB), 0) + off
            ki = lax.broadcasted_iota(jnp.int32, (QB, KB), 1)
            band = jnp.abs(qi - ki) <= WIN
            for h in range(HL):
                qh = q_bf[b * QB:(b + 1) * QB, h * DH:(h + 1) * DH]
                kh = kgvg[0, lo:lo + KB, h, :]
                s = lax.dot_general(qh, kh, (((1,), (1,)), ((), ())),
                                    preferred_element_type=jnp.float32) * SCALE
                s = jnp.where(band, s, -1e9)
                m = jnp.max(s, axis=1, keepdims=True)
                e = jnp.exp(s - m)
                w = (e / jnp.sum(e, axis=1, keepdims=True)).astype(jnp.bfloat16)
                vh = kgvg[1, lo:lo + KB, h, :]
                ctx_h = jnp.dot(w, vh, preferred_element_type=jnp.float32)
                ctx_bf[b * QB:(b + 1) * QB, h * DH:(h + 1) * DH] = (
                    ctx_h.astype(jnp.bfloat16))

        wo_bf = wo_ref[...].astype(jnp.bfloat16)
        partial = jnp.dot(ctx_bf[...], wo_bf,
                          preferred_element_type=jnp.float32)

        @pl.when(my == 0)
        def _():
            for r in kv_sends0:
                r.wait_send()

        @pl.when(my == 1)
        def _():
            for r in kv_sends1:
                r.wait_send()
            fwd1.wait_send()

        @pl.when(my == 3)
        def _():
            fwd3.wait_send()

        acc = partial
        for step, mask in enumerate((1, 3)):
            partner = my ^ mask
            ar_send[0] = acc.astype(jnp.bfloat16)
            r = pltpu.make_async_remote_copy(
                src_ref=ar_send.at[0], dst_ref=ar_recv.at[step],
                send_sem=ar_send_sems.at[step],
                recv_sem=ar_recv_sems.at[step],
                device_id=(partner,), device_id_type=pl.DeviceIdType.MESH)
            r.start()
            r.wait()
            acc = acc + ar_recv[step].astype(jnp.float32)

        out_ref[0, :, :] = acc

    return pl.pallas_call(
        body,
        out_shape=jax.ShapeDtypeStruct((1, SQ, D_MODEL), jnp.float32),
        in_specs=[
            pl.BlockSpec(memory_space=pltpu.VMEM),
            pl.BlockSpec(memory_space=pltpu.VMEM),
            pl.BlockSpec(memory_space=pl.ANY),
            pl.BlockSpec(memory_space=pl.ANY),
            pl.BlockSpec(memory_space=pltpu.VMEM),
        ],
        out_specs=pl.BlockSpec(memory_space=pltpu.VMEM),
        scratch_shapes=[
            pltpu.VMEM((3, 2, SQ, HL, DH), jnp.bfloat16),
            pltpu.VMEM((1, SQ, HL, DH), jnp.float32),
            pltpu.VMEM((2, KV_WIN, HL, DH), jnp.bfloat16),
            pltpu.VMEM((SQ, D_LOC), jnp.bfloat16),
            pltpu.VMEM((SQ, D_LOC), jnp.bfloat16),
            pltpu.VMEM((1, SQ, D_MODEL), jnp.bfloat16),
            pltpu.VMEM((2, SQ, D_MODEL), jnp.bfloat16),
            pltpu.VMEM((SQ, HL, DH), jnp.bfloat16),
            pltpu.SemaphoreType.DMA((4,)),
            pltpu.SemaphoreType.DMA((4,)),
            pltpu.SemaphoreType.DMA((2,)),
            pltpu.SemaphoreType.DMA((2,)),
            pltpu.SemaphoreType.DMA((2,)),
            pltpu.SemaphoreType.DMA((1,)),
            pltpu.SemaphoreType.DMA((1,)),
        ],
        compiler_params=pltpu.CompilerParams(
            collective_id=0, vmem_limit_bytes=60 * 1024 * 1024),
    )(x, Wq, K_ext, V_ext, Wo)


# device time: 134613 ns/iter; 1.9535x vs baseline; 1.2424x over previous
import jax
import jax.numpy as jnp
from jax import lax
from jax.experimental import pallas as pl
from jax.experimental.pallas import tpu as pltpu

N_DEV = 4
SQ = 1024
HL = 8
DH = 128
WIN = 128
KV_WIN = SQ + WIN
QB = 256
KB = 512
D_LOC = HL * DH
D_MODEL = 1024
SCALE = 0.08838834764831843


def kernel(x, Wq, K_ext, V_ext, Wo):
    def body(x_ref, wq_ref, k_ref, v_ref, wo_ref, out_ref,
             kv_bf, stag, kgvg, q_bf, ctx_bf, ar_send, ar_recv, relay_buf,
             kv_send_sems, kv_recv_sems, ar_send_sems, ar_recv_sems,
             copy_sems, relay_sems, fwd_send_sems):
        my = lax.axis_index("i")

        bsem = pltpu.get_barrier_semaphore()
        for j in range(N_DEV):
            @pl.when(my != j)
            def _(j=j):
                pl.semaphore_signal(bsem, inc=1, device_id=(j,),
                                    device_id_type=pl.DeviceIdType.MESH)
        pl.semaphore_wait(bsem, N_DEV - 1)

        def convert(slot, rows, heads0):
            for t, src in enumerate((k_ref, v_ref)):
                cp = pltpu.make_async_copy(
                    src.at[0, 0:rows, heads0:heads0 + HL, :],
                    stag.at[0, 0:rows], copy_sems.at[t])
                cp.start()
                cp.wait()
                kv_bf[slot, t, 0:rows] = stag[0, 0:rows].astype(jnp.bfloat16)

        def local_fill(rows, my_heads, dst_sl):
            for t, src in enumerate((k_ref, v_ref)):
                cp = pltpu.make_async_copy(
                    src.at[0, 0:rows, my_heads * HL:(my_heads + 1) * HL, :],
                    stag.at[0, 0:rows], copy_sems.at[t])
                cp.start()
                cp.wait()
                kgvg[t, dst_sl] = stag[0, 0:rows].astype(jnp.bfloat16)

        kv_sends0 = [
            pltpu.make_async_remote_copy(
                src_ref=kv_bf.at[2, 0], dst_ref=relay_buf,
                send_sem=kv_send_sems.at[2], recv_sem=relay_sems.at[0],
                device_id=(1,), device_id_type=pl.DeviceIdType.MESH),
            pltpu.make_async_remote_copy(
                src_ref=kv_bf.at[2, 1], dst_ref=relay_buf,
                send_sem=kv_send_sems.at[3], recv_sem=relay_sems.at[0],
                device_id=(3,), device_id_type=pl.DeviceIdType.MESH),
            pltpu.make_async_remote_copy(
                src_ref=kv_bf.at[0], dst_ref=kgvg.at[:, 0:SQ],
                send_sem=kv_send_sems.at[0], recv_sem=kv_recv_sems.at[0],
                device_id=(1,), device_id_type=pl.DeviceIdType.MESH),
            pltpu.make_async_remote_copy(
                src_ref=kv_bf.at[1], dst_ref=kgvg.at[:, 0:SQ],
                send_sem=kv_send_sems.at[1], recv_sem=kv_recv_sems.at[0],
                device_id=(3,), device_id_type=pl.DeviceIdType.MESH),
        ]
        kv_sends1 = [pltpu.make_async_remote_copy(
            src_ref=kv_bf.at[jj, :, 0:WIN],
            dst_ref=kgvg.at[:, SQ:KV_WIN],
            send_sem=kv_send_sems.at[jj],
            recv_sem=kv_recv_sems.at[1],
            device_id=(j,), device_id_type=pl.DeviceIdType.MESH)
            for jj, j in enumerate((0, 2, 3))]
        fwd1 = pltpu.make_async_remote_copy(
            src_ref=relay_buf, dst_ref=kgvg.at[0, 0:SQ],
            send_sem=fwd_send_sems.at[0], recv_sem=kv_recv_sems.at[2],
            device_id=(2,), device_id_type=pl.DeviceIdType.MESH)
        fwd3 = pltpu.make_async_remote_copy(
            src_ref=relay_buf, dst_ref=kgvg.at[1, 0:SQ],
            send_sem=fwd_send_sems.at[0], recv_sem=kv_recv_sems.at[3],
            device_id=(2,), device_id_type=pl.DeviceIdType.MESH)

        @pl.when(my == 0)
        def _():
            convert(2, SQ, 2 * HL)
            kv_sends0[0].start()
            kv_sends0[1].start()
            convert(0, SQ, 1 * HL)
            kv_sends0[2].start()
            convert(1, SQ, 3 * HL)
            kv_sends0[3].start()
            local_fill(SQ, 0, slice(0, SQ))

        @pl.when(my == 1)
        def _():
            for jj, j in enumerate((0, 2, 3)):
                convert(jj, WIN, j * HL)
                kv_sends1[jj].start()
            local_fill(WIN, 1, slice(SQ, KV_WIN))

        x_bf = x_ref[0].astype(jnp.bfloat16)
        wq_bf = wq_ref[...].astype(jnp.bfloat16)
        q = jnp.dot(x_bf, wq_bf, preferred_element_type=jnp.float32)
        q_bf[...] = q.astype(jnp.bfloat16)

        @pl.when((my == 1) | (my == 3))
        def _():
            pltpu.make_async_remote_copy(
                src_ref=relay_buf, dst_ref=relay_buf,
                send_sem=fwd_send_sems.at[0],
                recv_sem=relay_sems.at[0],
                device_id=(0,), device_id_type=pl.DeviceIdType.MESH,
            ).wait_recv()

        @pl.when(my == 1)
        def _():
            fwd1.start()

        @pl.when(my == 3)
        def _():
            fwd3.start()

        @pl.when((my == 1) | (my == 3))
        def _():
            pltpu.make_async_remote_copy(
                src_ref=kgvg.at[:, 0:SQ], dst_ref=kgvg.at[:, 0:SQ],
                send_sem=kv_send_sems.at[0],
                recv_sem=kv_recv_sems.at[0],
                device_id=(0,), device_id_type=pl.DeviceIdType.MESH,
            ).wait_recv()

        @pl.when(my == 2)
        def _():
            for t in (2, 3):
                pltpu.make_async_remote_copy(
                    src_ref=kgvg.at[t - 2, 0:SQ], dst_ref=kgvg.at[t - 2, 0:SQ],
                    send_sem=kv_send_sems.at[0],
                    recv_sem=kv_recv_sems.at[t],
                    device_id=(0,), device_id_type=pl.DeviceIdType.MESH,
                ).wait_recv()

        @pl.when(my != 1)
        def _():
            pltpu.make_async_remote_copy(
                src_ref=kgvg.at[:, SQ:KV_WIN], dst_ref=kgvg.at[:, SQ:KV_WIN],
                send_sem=kv_send_sems.at[0],
                recv_sem=kv_recv_sems.at[1],
                device_id=(1,), device_id_type=pl.DeviceIdType.MESH,
            ).wait_recv()

        wo_bf = wo_ref[...].astype(jnp.bfloat16)
        p1 = my ^ 1
        p2 = my ^ 3
        NB = SQ // QB

        def make_ex(step, b, partner):
            return pltpu.make_async_remote_copy(
                src_ref=ar_send.at[step, b], dst_ref=ar_recv.at[step, b],
                send_sem=ar_send_sems.at[step * NB + b],
                recv_sem=ar_recv_sems.at[step * NB + b],
                device_id=(partner,), device_id_type=pl.DeviceIdType.MESH)

        partials = [None] * NB
        sums = [None] * NB
        ex0 = [None] * NB
        ex1 = [None] * NB

        def start_step1(b):
            ex0[b].wait()
            sums[b] = partials[b] + ar_recv[0, b].astype(jnp.float32)
            ar_send[1, b] = sums[b].astype(jnp.bfloat16)
            ex1[b] = make_ex(1, b, p2)
            ex1[b].start()

        def finish_block(b):
            ex1[b].wait()
            out_ref[0, b * QB:(b + 1) * QB, :] = (
                sums[b] + ar_recv[1, b].astype(jnp.float32))

        for b in range(NB):
            lo = max(0, QB * b - WIN)
            off = QB * b - lo
            qi = lax.broadcasted_iota(jnp.int32, (QB, KB), 0) + off
            ki = lax.broadcasted_iota(jnp.int32, (QB, KB), 1)
            band = jnp.abs(qi - ki) <= WIN
            for h in range(HL):
                qh = q_bf[b * QB:(b + 1) * QB, h * DH:(h + 1) * DH]
                kh = kgvg[0, lo:lo + KB, h, :]
                s = lax.dot_general(qh, kh, (((1,), (1,)), ((), ())),
                                    preferred_element_type=jnp.float32) * SCALE
                s = jnp.where(band, s, -1e9)
                m = jnp.max(s, axis=1, keepdims=True)
                e = jnp.exp(s - m)
                w = (e / jnp.sum(e, axis=1, keepdims=True)).astype(jnp.bfloat16)
                vh = kgvg[1, lo:lo + KB, h, :]
                ctx_h = jnp.dot(w, vh, preferred_element_type=jnp.float32)
                ctx_bf[b * QB:(b + 1) * QB, h * DH:(h + 1) * DH] = (
                    ctx_h.astype(jnp.bfloat16))
            partials[b] = jnp.dot(ctx_bf[b * QB:(b + 1) * QB, :], wo_bf,
                                  preferred_element_type=jnp.float32)
            ar_send[0, b] = partials[b].astype(jnp.bfloat16)
            ex0[b] = make_ex(0, b, p1)
            ex0[b].start()
            if b >= 1:
                start_step1(b - 1)
            if b >= 2:
                finish_block(b - 2)

        start_step1(NB - 1)
        finish_block(NB - 2)
        finish_block(NB - 1)

        @pl.when(my == 0)
        def _():
            for r in kv_sends0:
                r.wait_send()

        @pl.when(my == 1)
        def _():
            for r in kv_sends1:
                r.wait_send()
            fwd1.wait_send()

        @pl.when(my == 3)
        def _():
            fwd3.wait_send()

    return pl.pallas_call(
        body,
        out_shape=jax.ShapeDtypeStruct((1, SQ, D_MODEL), jnp.float32),
        in_specs=[
            pl.BlockSpec(memory_space=pltpu.VMEM),
            pl.BlockSpec(memory_space=pltpu.VMEM),
            pl.BlockSpec(memory_space=pl.ANY),
            pl.BlockSpec(memory_space=pl.ANY),
            pl.BlockSpec(memory_space=pltpu.VMEM),
        ],
        out_specs=pl.BlockSpec(memory_space=pltpu.VMEM),
        scratch_shapes=[
            pltpu.VMEM((3, 2, SQ, HL, DH), jnp.bfloat16),
            pltpu.VMEM((1, SQ, HL, DH), jnp.float32),
            pltpu.VMEM((2, KV_WIN, HL, DH), jnp.bfloat16),
            pltpu.VMEM((SQ, D_LOC), jnp.bfloat16),
            pltpu.VMEM((SQ, D_LOC), jnp.bfloat16),
            pltpu.VMEM((2, 4, QB, D_MODEL), jnp.bfloat16),
            pltpu.VMEM((2, 4, QB, D_MODEL), jnp.bfloat16),
            pltpu.VMEM((SQ, HL, DH), jnp.bfloat16),
            pltpu.SemaphoreType.DMA((4,)),
            pltpu.SemaphoreType.DMA((4,)),
            pltpu.SemaphoreType.DMA((8,)),
            pltpu.SemaphoreType.DMA((8,)),
            pltpu.SemaphoreType.DMA((2,)),
            pltpu.SemaphoreType.DMA((1,)),
            pltpu.SemaphoreType.DMA((1,)),
        ],
        compiler_params=pltpu.CompilerParams(
            collective_id=0, vmem_limit_bytes=60 * 1024 * 1024),
    )(x, Wq, K_ext, V_ext, Wo)


# device time: 121967 ns/iter; 2.1560x vs baseline; 1.1037x over previous
import jax
import jax.numpy as jnp
from jax import lax
from jax.experimental import pallas as pl
from jax.experimental.pallas import tpu as pltpu

N_DEV = 4
SQ = 1024
HL = 8
DH = 128
WIN = 128
KV_WIN = SQ + WIN
QB = 256
KB = 512
CA = 640
D_LOC = HL * DH
D_MODEL = 1024
SCALE = 0.08838834764831843


def kernel(x, Wq, K_ext, V_ext, Wo):
    def body(x_ref, wq_ref, k_ref, v_ref, wo_ref, out_ref,
             kv_bf, stag, kgvg, q_bf, ctx_bf, ar_send, ar_recv, relay_buf,
             kv_send_sems, kv_recv_sems, ar_send_sems, ar_recv_sems,
             copy_sems, relay_sems, fwd_send_sems):
        my = lax.axis_index("i")

        bsem = pltpu.get_barrier_semaphore()
        for j in range(N_DEV):
            @pl.when(my != j)
            def _(j=j):
                pl.semaphore_signal(bsem, inc=1, device_id=(j,),
                                    device_id_type=pl.DeviceIdType.MESH)
        pl.semaphore_wait(bsem, N_DEV - 1)

        def convert1(slot, t, r0, r1, heads0):
            src = (k_ref, v_ref)[t]
            cp = pltpu.make_async_copy(
                src.at[0, r0:r1, heads0:heads0 + HL, :],
                stag.at[0, 0:r1 - r0], copy_sems.at[0])
            cp.start()
            cp.wait()
            kv_bf[slot, t, r0:r1] = stag[0, 0:r1 - r0].astype(jnp.bfloat16)

        def local_fill(rows, my_heads, dst_sl):
            for t, src in enumerate((k_ref, v_ref)):
                cp = pltpu.make_async_copy(
                    src.at[0, 0:rows, my_heads * HL:(my_heads + 1) * HL, :],
                    stag.at[0, 0:rows], copy_sems.at[0])
                cp.start()
                cp.wait()
                kgvg[t, dst_sl] = stag[0, 0:rows].astype(jnp.bfloat16)

        def rc(src_ref, dst_ref, send_sem, recv_sem, dev):
            return pltpu.make_async_remote_copy(
                src_ref=src_ref, dst_ref=dst_ref, send_sem=send_sem,
                recv_sem=recv_sem, device_id=(dev,),
                device_id_type=pl.DeviceIdType.MESH)

        kv_sends0 = [
            rc(kv_bf.at[2, 0, 0:CA], relay_buf.at[0:CA],
               kv_send_sems.at[0], relay_sems.at[0], 1),
            rc(kv_bf.at[2, 1, 0:CA], relay_buf.at[0:CA],
               kv_send_sems.at[1], relay_sems.at[0], 3),
            rc(kv_bf.at[2, 0, CA:SQ], relay_buf.at[CA:SQ],
               kv_send_sems.at[2], relay_sems.at[1], 1),
            rc(kv_bf.at[2, 1, CA:SQ], relay_buf.at[CA:SQ],
               kv_send_sems.at[3], relay_sems.at[1], 3),
            rc(kv_bf.at[0, :, 0:CA], kgvg.at[:, 0:CA],
               kv_send_sems.at[4], kv_recv_sems.at[0], 1),
            rc(kv_bf.at[1, :, 0:CA], kgvg.at[:, 0:CA],
               kv_send_sems.at[5], kv_recv_sems.at[0], 3),
            rc(kv_bf.at[0, :, CA:SQ], kgvg.at[:, CA:SQ],
               kv_send_sems.at[6], kv_recv_sems.at[4], 1),
            rc(kv_bf.at[1, :, CA:SQ], kgvg.at[:, CA:SQ],
               kv_send_sems.at[7], kv_recv_sems.at[4], 3),
        ]
        kv_sends1 = [rc(kv_bf.at[jj, :, 0:WIN], kgvg.at[:, SQ:KV_WIN],
                        kv_send_sems.at[jj], kv_recv_sems.at[1], j)
                     for jj, j in enumerate((0, 2, 3))]
        fwds = {
            1: [rc(relay_buf.at[0:CA], kgvg.at[0, 0:CA],
                   fwd_send_sems.at[0], kv_recv_sems.at[2], 2),
                rc(relay_buf.at[CA:SQ], kgvg.at[0, CA:SQ],
                   fwd_send_sems.at[1], kv_recv_sems.at[5], 2)],
            3: [rc(relay_buf.at[0:CA], kgvg.at[1, 0:CA],
                   fwd_send_sems.at[0], kv_recv_sems.at[3], 2),
                rc(relay_buf.at[CA:SQ], kgvg.at[1, CA:SQ],
                   fwd_send_sems.at[1], kv_recv_sems.at[6], 2)],
        }

        @pl.when(my == 0)
        def _():
            convert1(2, 0, 0, CA, 2 * HL)
            kv_sends0[0].start()
            convert1(2, 1, 0, CA, 2 * HL)
            kv_sends0[1].start()
            convert1(2, 0, CA, SQ, 2 * HL)
            kv_sends0[2].start()
            convert1(2, 1, CA, SQ, 2 * HL)
            kv_sends0[3].start()
            convert1(0, 0, 0, CA, 1 * HL)
            convert1(0, 1, 0, CA, 1 * HL)
            kv_sends0[4].start()
            convert1(1, 0, 0, CA, 3 * HL)
            convert1(1, 1, 0, CA, 3 * HL)
            kv_sends0[5].start()
            convert1(0, 0, CA, SQ, 1 * HL)
            convert1(0, 1, CA, SQ, 1 * HL)
            kv_sends0[6].start()
            convert1(1, 0, CA, SQ, 3 * HL)
            convert1(1, 1, CA, SQ, 3 * HL)
            kv_sends0[7].start()
            local_fill(SQ, 0, slice(0, SQ))

        @pl.when(my == 1)
        def _():
            for jj, j in enumerate((0, 2, 3)):
                convert1(jj, 0, 0, WIN, j * HL)
                convert1(jj, 1, 0, WIN, j * HL)
                kv_sends1[jj].start()
            local_fill(WIN, 1, slice(SQ, KV_WIN))

        x_bf = x_ref[0].astype(jnp.bfloat16)
        wq_bf = wq_ref[...].astype(jnp.bfloat16)
        q = jnp.dot(x_bf, wq_bf, preferred_element_type=jnp.float32)
        q_bf[...] = q.astype(jnp.bfloat16)

        def wait_bytes(dst_ref, recv_sem):
            pltpu.make_async_remote_copy(
                src_ref=dst_ref, dst_ref=dst_ref,
                send_sem=fwd_send_sems.at[0], recv_sem=recv_sem,
                device_id=(0,), device_id_type=pl.DeviceIdType.MESH,
            ).wait_recv()

        for relayer in (1, 3):
            @pl.when(my == relayer)
            def _(relayer=relayer):
                wait_bytes(relay_buf.at[0:CA], relay_sems.at[0])
                fwds[relayer][0].start()
                wait_bytes(relay_buf.at[CA:SQ], relay_sems.at[1])
                fwds[relayer][1].start()

        wo_bf = wo_ref[...].astype(jnp.bfloat16)
        p1 = my ^ 1
        p2 = my ^ 3
        NB = SQ // QB

        def make_ex(step, b, partner):
            return pltpu.make_async_remote_copy(
                src_ref=ar_send.at[step, b], dst_ref=ar_recv.at[step, b],
                send_sem=ar_send_sems.at[step * NB + b],
                recv_sem=ar_recv_sems.at[step * NB + b],
                device_id=(partner,), device_id_type=pl.DeviceIdType.MESH)

        partials = [None] * NB
        sums = [None] * NB
        ex0 = [None] * NB
        ex1 = [None] * NB

        def start_step1(b):
            ex0[b].wait()
            sums[b] = partials[b] + ar_recv[0, b].astype(jnp.float32)
            ar_send[1, b] = sums[b].astype(jnp.bfloat16)
            ex1[b] = make_ex(1, b, p2)
            ex1[b].start()

        def finish_block(b):
            ex1[b].wait()
            out_ref[0, b * QB:(b + 1) * QB, :] = (
                sums[b] + ar_recv[1, b].astype(jnp.float32))

        for b in range(NB):
            if b == 0 or b == 2:
                direct_sem = kv_recv_sems.at[0 if b == 0 else 4]
                rel_k = kv_recv_sems.at[2 if b == 0 else 5]
                rel_v = kv_recv_sems.at[3 if b == 0 else 6]
                rows = slice(0, CA) if b == 0 else slice(CA, SQ)

                @pl.when((my == 1) | (my == 3))
                def _(direct_sem=direct_sem, rows=rows):
                    wait_bytes(kgvg.at[:, rows], direct_sem)

                @pl.when(my == 2)
                def _(rel_k=rel_k, rel_v=rel_v, rows=rows):
                    wait_bytes(kgvg.at[0, rows], rel_k)
                    wait_bytes(kgvg.at[1, rows], rel_v)
            if b == 3:
                @pl.when(my != 1)
                def _():
                    wait_bytes(kgvg.at[:, SQ:KV_WIN], kv_recv_sems.at[1])

            lo = max(0, QB * b - WIN)
            off = QB * b - lo
            qi = lax.broadcasted_iota(jnp.int32, (QB, KB), 0) + off
            ki = lax.broadcasted_iota(jnp.int32, (QB, KB), 1)
            band = jnp.abs(qi - ki) <= WIN
            for h in range(HL):
                qh = q_bf[b * QB:(b + 1) * QB, h * DH:(h + 1) * DH]
                kh = kgvg[0, lo:lo + KB, h, :]
                s = lax.dot_general(qh, kh, (((1,), (1,)), ((), ())),
                                    preferred_element_type=jnp.float32) * SCALE
                s = jnp.where(band, s, -1e9)
                m = jnp.max(s, axis=1, keepdims=True)
                e = jnp.exp(s - m)
                w = (e / jnp.sum(e, axis=1, keepdims=True)).astype(jnp.bfloat16)
                vh = kgvg[1, lo:lo + KB, h, :]
                ctx_h = jnp.dot(w, vh, preferred_element_type=jnp.float32)
                ctx_bf[b * QB:(b + 1) * QB, h * DH:(h + 1) * DH] = (
                    ctx_h.astype(jnp.bfloat16))
            partials[b] = jnp.dot(ctx_bf[b * QB:(b + 1) * QB, :], wo_bf,
                                  preferred_element_type=jnp.float32)
            ar_send[0, b] = partials[b].astype(jnp.bfloat16)
            ex0[b] = make_ex(0, b, p1)
            ex0[b].start()
            if b >= 1:
                start_step1(b - 1)
            if b >= 2:
                finish_block(b - 2)

        start_step1(NB - 1)
        finish_block(NB - 2)
        finish_block(NB - 1)

        @pl.when(my == 0)
        def _():
            for r in kv_sends0:
                r.wait_send()

        @pl.when(my == 1)
        def _():
            for r in kv_sends1:
                r.wait_send()

        for relayer in (1, 3):
            @pl.when(my == relayer)
            def _(relayer=relayer):
                for r in fwds[relayer]:
                    r.wait_send()

    return pl.pallas_call(
        body,
        out_shape=jax.ShapeDtypeStruct((1, SQ, D_MODEL), jnp.float32),
        in_specs=[
            pl.BlockSpec(memory_space=pltpu.VMEM),
            pl.BlockSpec(memory_space=pltpu.VMEM),
            pl.BlockSpec(memory_space=pl.ANY),
            pl.BlockSpec(memory_space=pl.ANY),
            pl.BlockSpec(memory_space=pltpu.VMEM),
        ],
        out_specs=pl.BlockSpec(memory_space=pltpu.VMEM),
        scratch_shapes=[
            pltpu.VMEM((3, 2, SQ, HL, DH), jnp.bfloat16),
            pltpu.VMEM((1, SQ, HL, DH), jnp.float32),
            pltpu.VMEM((2, KV_WIN, HL, DH), jnp.bfloat16),
            pltpu.VMEM((SQ, D_LOC), jnp.bfloat16),
            pltpu.VMEM((SQ, D_LOC), jnp.bfloat16),
            pltpu.VMEM((2, 4, QB, D_MODEL), jnp.bfloat16),
            pltpu.VMEM((2, 4, QB, D_MODEL), jnp.bfloat16),
            pltpu.VMEM((SQ, HL, DH), jnp.bfloat16),
            pltpu.SemaphoreType.DMA((8,)),
            pltpu.SemaphoreType.DMA((7,)),
            pltpu.SemaphoreType.DMA((8,)),
            pltpu.SemaphoreType.DMA((8,)),
            pltpu.SemaphoreType.DMA((1,)),
            pltpu.SemaphoreType.DMA((2,)),
            pltpu.SemaphoreType.DMA((2,)),
        ],
        compiler_params=pltpu.CompilerParams(
            collective_id=0, vmem_limit_bytes=60 * 1024 * 1024),
    )(x, Wq, K_ext, V_ext, Wo)
